# Initial kernel scaffold; baseline (speedup 1.0000x reference)
import jax
import jax.numpy as jnp
from jax import lax
from jax.experimental import pallas as pl
from jax.experimental.pallas import tpu as pltpu
from jax.experimental.pallas import tpu_sc as plsc

N = 10000
E = 160000
F = 256
HALF = 128
NSC = 2
NTEC = 16
LANES = 16

N_PAD = 10240
ROWS_PER_TEC = N_PAD // NTEC
EDGE_BLK = 128
EPT = 10112
NB = EPT // EDGE_BLK
E_PAD = EPT * NTEC

BN = 1000


def _sc_degree_body(src_hbm, dst_hbm, out_hbm, idx_v, hist_v):
    c = lax.axis_index("c")
    s = lax.axis_index("s")

    @pl.loop(0, N_PAD // LANES)
    def _zero(i):
        hist_v[pl.ds(i * LANES, LANES)] = jnp.zeros((LANES,), jnp.float32)

    @pl.when(c == 0)
    def _():
        pltpu.sync_copy(src_hbm.at[s], idx_v)

    @pl.when(c == 1)
    def _():
        pltpu.sync_copy(dst_hbm.at[s], idx_v)

    ones = jnp.ones((LANES,), jnp.float32)

    @pl.loop(0, NB)
    def _blocks(b):
        for j in range(EDGE_BLK // LANES):
            idx = idx_v[b, pl.ds(j * LANES, LANES)]
            plsc.addupdate_scatter(hist_v, [idx], ones)

    pltpu.sync_copy(hist_v, out_hbm.at[c, s])


def _sc_degree(src_p, dst_p):
    mesh = plsc.VectorSubcoreMesh(core_axis_name="c", subcore_axis_name="s")
    return pl.kernel(
        _sc_degree_body,
        out_type=jax.ShapeDtypeStruct((NSC, NTEC, N_PAD), jnp.float32),
        mesh=mesh,
        scratch_types=[
            pltpu.VMEM((NB, EDGE_BLK), jnp.int32),
            pltpu.VMEM((N_PAD,), jnp.float32),
        ],
    )(src_p, dst_p)


def _sc_agg_body(hs_a, hs_b, src_hbm, dst_hbm, out_a, out_b,
                 idx_s, idx_d, rows_v, stage_v, zero_v, agg_sh, gsem):
    c = lax.axis_index("c")
    s = lax.axis_index("s")

    for r in range(64):
        for j in range(HALF // LANES):
            zero_v[r, pl.ds(j * LANES, LANES)] = jnp.zeros((LANES,), jnp.float32)
    for r in range(ROWS_PER_TEC // 64):
        pltpu.sync_copy(zero_v, agg_sh.at[pl.ds(s * ROWS_PER_TEC + r * 64, 64)])

    pltpu.sync_copy(src_hbm.at[s], idx_s)
    pltpu.sync_copy(dst_hbm.at[s], idx_d)
    plsc.subcore_barrier()

    def process(table):
        @pl.loop(0, NB)
        def _blocks(b):
            pltpu.async_copy(table.at[idx_s.at[b]], rows_v, gsem).wait()
            pltpu.sync_copy(rows_v, agg_sh.at[idx_d.at[b]], add=True)

    @pl.when(c == 0)
    def _():
        process(hs_a)

    @pl.when(c == 1)
    def _():
        process(hs_b)

    plsc.subcore_barrier()

    def writeout(out):
        for r in range(2):
            off = s * ROWS_PER_TEC + r * (ROWS_PER_TEC // 2)
            pltpu.sync_copy(agg_sh.at[pl.ds(off, ROWS_PER_TEC // 2)], stage_v)
            pltpu.sync_copy(stage_v, out.at[pl.ds(off, ROWS_PER_TEC // 2)])

    @pl.when(c == 0)
    def _():
        writeout(out_a)

    @pl.when(c == 1)
    def _():
        writeout(out_b)


def _sc_agg(hs_a, hs_b, src_p, dst_p):
    mesh = plsc.VectorSubcoreMesh(core_axis_name="c", subcore_axis_name="s")
    return pl.kernel(
        _sc_agg_body,
        out_type=(jax.ShapeDtypeStruct((N_PAD, HALF), jnp.float32),
                  jax.ShapeDtypeStruct((N_PAD, HALF), jnp.float32)),
        mesh=mesh,
        scratch_types=[
            pltpu.VMEM((NB, EDGE_BLK), jnp.int32),
            pltpu.VMEM((NB, EDGE_BLK), jnp.int32),
            pltpu.VMEM((EDGE_BLK, HALF), jnp.float32),
            pltpu.VMEM((ROWS_PER_TEC // 2, HALF), jnp.float32),
            pltpu.VMEM((64, HALF), jnp.float32),
            pltpu.VMEM_SHARED((N_PAD, HALF), jnp.float32),
            pltpu.SemaphoreType.DMA,
        ],
    )(hs_a, hs_b, src_p, dst_p)


def _tc_embed_body(x_ref, w_ref, hist_ref, out_a, out_b):
    h = jnp.dot(x_ref[...], w_ref[...], preferred_element_type=jnp.float32)
    deg = jnp.sum(hist_ref[0], axis=1, keepdims=True)
    ns = lax.rsqrt(jnp.maximum(deg, 1.0))
    hs = h * ns
    out_a[...] = hs[:, :HALF]
    out_b[...] = hs[:, HALF:]


def _tc_embed(x, w0, hists_t):
    return pl.pallas_call(
        _tc_embed_body,
        grid=(N // BN,),
        in_specs=[
            pl.BlockSpec((BN, F), lambda i: (i, 0)),
            pl.BlockSpec((F, F), lambda i: (0, 0)),
            pl.BlockSpec((NSC, BN, NTEC), lambda i: (0, i, 0)),
        ],
        out_specs=[
            pl.BlockSpec((BN, HALF), lambda i: (i, 0)),
            pl.BlockSpec((BN, HALF), lambda i: (i, 0)),
        ],
        out_shape=[jax.ShapeDtypeStruct((N_PAD, HALF), jnp.float32),
                   jax.ShapeDtypeStruct((N_PAD, HALF), jnp.float32)],
    )(x, w0, hists_t)


def _tc_layer_body(agg_a, agg_b, hist_ref, w_ref, b_ref, out_a, out_b):
    a = jnp.concatenate([agg_a[...], agg_b[...]], axis=1)
    deg_in = jnp.sum(hist_ref[1], axis=1, keepdims=True)
    nd = lax.rsqrt(jnp.maximum(deg_in, 1.0))
    h = jnp.dot(a * nd, w_ref[...], preferred_element_type=jnp.float32)
    h = jnp.maximum(h + b_ref[...], 0.0)
    deg_out = jnp.sum(hist_ref[0], axis=1, keepdims=True)
    ns = lax.rsqrt(jnp.maximum(deg_out, 1.0))
    hs = h * ns
    out_a[...] = hs[:, :HALF]
    out_b[...] = hs[:, HALF:]


def _tc_layer(agg_a, agg_b, hists_t, w, b):
    return pl.pallas_call(
        _tc_layer_body,
        grid=(N // BN,),
        in_specs=[
            pl.BlockSpec((BN, HALF), lambda i: (i, 0)),
            pl.BlockSpec((BN, HALF), lambda i: (i, 0)),
            pl.BlockSpec((NSC, BN, NTEC), lambda i: (0, i, 0)),
            pl.BlockSpec((F, F), lambda i: (0, 0)),
            pl.BlockSpec((1, F), lambda i: (0, 0)),
        ],
        out_specs=[
            pl.BlockSpec((BN, HALF), lambda i: (i, 0)),
            pl.BlockSpec((BN, HALF), lambda i: (i, 0)),
        ],
        out_shape=[jax.ShapeDtypeStruct((N_PAD, HALF), jnp.float32),
                   jax.ShapeDtypeStruct((N_PAD, HALF), jnp.float32)],
    )(agg_a, agg_b, hists_t, w, b)


def _tc_final_body(agg_a, agg_b, hist_ref, w_ref, b_ref, out_ref):
    i = pl.program_id(0)

    @pl.when(i == 0)
    def _():
        out_ref[...] = jnp.full((1, F), -jnp.inf, jnp.float32)

    a = jnp.concatenate([agg_a[...], agg_b[...]], axis=1)
    deg_in = jnp.sum(hist_ref[1], axis=1, keepdims=True)
    nd = lax.rsqrt(jnp.maximum(deg_in, 1.0))
    h = jnp.dot(a * nd, w_ref[...], preferred_element_type=jnp.float32)
    h = jnp.maximum(h + b_ref[...], 0.0)
    out_ref[...] = jnp.maximum(out_ref[...], jnp.max(h, axis=0, keepdims=True))


def _tc_final(agg_a, agg_b, hists_t, w, b):
    return pl.pallas_call(
        _tc_final_body,
        grid=(N // BN,),
        in_specs=[
            pl.BlockSpec((BN, HALF), lambda i: (i, 0)),
            pl.BlockSpec((BN, HALF), lambda i: (i, 0)),
            pl.BlockSpec((NSC, BN, NTEC), lambda i: (0, i, 0)),
            pl.BlockSpec((F, F), lambda i: (0, 0)),
            pl.BlockSpec((1, F), lambda i: (0, 0)),
        ],
        out_specs=pl.BlockSpec((1, F), lambda i: (0, 0)),
        out_shape=jax.ShapeDtypeStruct((1, F), jnp.float32),
        compiler_params=pltpu.CompilerParams(
            dimension_semantics=("arbitrary",)),
    )(agg_a, agg_b, hists_t, w, b)


@jax.jit
def kernel(x, edge_index, W0, W1, b1, W2, b2):
    src = edge_index[0]
    dst = edge_index[1]
    pad = jnp.full((E_PAD - E,), N, jnp.int32)
    src_p = jnp.concatenate([src, pad]).reshape(NTEC, NB, EDGE_BLK)
    dst_p = jnp.concatenate([dst, pad]).reshape(NTEC, NB, EDGE_BLK)

    hists = _sc_degree(src_p, dst_p)
    hists_t = hists.transpose(0, 2, 1)

    hs_a, hs_b = _tc_embed(x, W0, hists_t)
    agg_a, agg_b = _sc_agg(hs_a, hs_b, src_p, dst_p)
    hs_a, hs_b = _tc_layer(agg_a, agg_b, hists_t, W1, b1.reshape(1, F))
    agg_a, agg_b = _sc_agg(hs_a, hs_b, src_p, dst_p)
    out = _tc_final(agg_a, agg_b, hists_t, W2, b2.reshape(1, F))
    return out.reshape(F)


# speedup vs baseline: 4.7886x; 4.7886x over previous
import jax
import jax.numpy as jnp
from jax import lax
from jax.experimental import pallas as pl
from jax.experimental.pallas import tpu as pltpu
from jax.experimental.pallas import tpu_sc as plsc

N = 10000
E = 160000
F = 256
QF = 64
NSC = 2
NTEC = 16
LANES = 16

N_PAD = 10240
ROWS_PER_TEC = N_PAD // NTEC
EDGE_BLK = 128
EPT = 10112
NB = EPT // EDGE_BLK
E_PAD = EPT * NTEC

BN = 1000


def _sc_degree_body(src_hbm, dst_hbm, out_hbm, idx_v, hist_v):
    c = lax.axis_index("c")
    s = lax.axis_index("s")

    @pl.loop(0, N_PAD // LANES)
    def _zero(i):
        hist_v[pl.ds(i * LANES, LANES)] = jnp.zeros((LANES,), jnp.float32)

    @pl.when(c == 0)
    def _():
        pltpu.sync_copy(src_hbm.at[s], idx_v)

    @pl.when(c == 1)
    def _():
        pltpu.sync_copy(dst_hbm.at[s], idx_v)

    ones = jnp.ones((LANES,), jnp.float32)

    @pl.loop(0, NB)
    def _blocks(b):
        for j in range(EDGE_BLK // LANES):
            idx = idx_v[b, pl.ds(j * LANES, LANES)]
            plsc.addupdate_scatter(hist_v, [idx], ones)

    pltpu.sync_copy(hist_v, out_hbm.at[c, s])


def _sc_degree(src_p, dst_p):
    mesh = plsc.VectorSubcoreMesh(core_axis_name="c", subcore_axis_name="s")
    return pl.kernel(
        _sc_degree_body,
        out_type=jax.ShapeDtypeStruct((NSC, NTEC, N_PAD), jnp.float32),
        mesh=mesh,
        scratch_types=[
            pltpu.VMEM((NB, EDGE_BLK), jnp.int32),
            pltpu.VMEM((N_PAD,), jnp.float32),
        ],
        compiler_params=pltpu.CompilerParams(needs_layout_passes=False),
    )(src_p, dst_p)


def _sc_agg_body(hs_q0, hs_q1, hs_q2, hs_q3, src_hbm, dst_hbm,
                 out_q0, out_q1, out_q2, out_q3,
                 idx_s, idx_d, rows_v, zero_v, agg_sh, gsem):
    c = lax.axis_index("c")
    s = lax.axis_index("s")

    for r in range(64):
        for j in range(QF // LANES):
            zero_v[r, pl.ds(j * LANES, LANES)] = jnp.zeros((LANES,), jnp.float32)

    pltpu.sync_copy(src_hbm.at[s], idx_s)
    pltpu.sync_copy(dst_hbm.at[s], idx_d)

    def one_pass(table, out):
        for r in range(ROWS_PER_TEC // 64):
            pltpu.sync_copy(zero_v,
                            agg_sh.at[pl.ds(s * ROWS_PER_TEC + r * 64, 64)])
        plsc.subcore_barrier()

        @pl.loop(0, NB)
        def _blocks(b):
            pltpu.async_copy(table.at[idx_s.at[b]], rows_v, gsem).wait()
            pltpu.sync_copy(rows_v, agg_sh.at[idx_d.at[b]], add=True)

        plsc.subcore_barrier()
        off = s * ROWS_PER_TEC
        pltpu.sync_copy(agg_sh.at[pl.ds(off, ROWS_PER_TEC)],
                        out.at[pl.ds(off, ROWS_PER_TEC)])
        plsc.subcore_barrier()

    @pl.when(c == 0)
    def _():
        one_pass(hs_q0, out_q0)
        one_pass(hs_q1, out_q1)

    @pl.when(c == 1)
    def _():
        one_pass(hs_q2, out_q2)
        one_pass(hs_q3, out_q3)


def _sc_agg(hs_q, src_p, dst_p):
    mesh = plsc.VectorSubcoreMesh(core_axis_name="c", subcore_axis_name="s")
    qshape = jax.ShapeDtypeStruct((N_PAD, QF), jnp.float32)
    return pl.kernel(
        _sc_agg_body,
        out_type=(qshape,) * 4,
        mesh=mesh,
        scratch_types=[
            pltpu.VMEM((NB, EDGE_BLK), jnp.int32),
            pltpu.VMEM((NB, EDGE_BLK), jnp.int32),
            pltpu.VMEM((EDGE_BLK, QF), jnp.float32),
            pltpu.VMEM((64, QF), jnp.float32),
            pltpu.VMEM_SHARED((N_PAD, QF), jnp.float32),
            pltpu.SemaphoreType.DMA,
        ],
        compiler_params=pltpu.CompilerParams(needs_layout_passes=False,
                                             use_tc_tiling_on_sc=False),
    )(*hs_q, src_p, dst_p)


def _split_store(hs, outs):
    for q in range(4):
        outs[q][...] = hs[:, q * QF:(q + 1) * QF]


def _tc_embed_body(x_ref, w_ref, hist_ref, *outs):
    h = jnp.dot(x_ref[...], w_ref[...], preferred_element_type=jnp.float32)
    deg = jnp.sum(hist_ref[0], axis=1, keepdims=True)
    ns = lax.rsqrt(jnp.maximum(deg, 1.0))
    _split_store(h * ns, outs)


_QSPECS = [pl.BlockSpec((BN, QF), lambda i: (i, 0)) for _ in range(4)]
_QSHAPES = [jax.ShapeDtypeStruct((N_PAD, QF), jnp.float32) for _ in range(4)]


def _tc_embed(x, w0, hists_t):
    return pl.pallas_call(
        _tc_embed_body,
        grid=(N // BN,),
        in_specs=[
            pl.BlockSpec((BN, F), lambda i: (i, 0)),
            pl.BlockSpec((F, F), lambda i: (0, 0)),
            pl.BlockSpec((NSC, BN, NTEC), lambda i: (0, i, 0)),
        ],
        out_specs=_QSPECS,
        out_shape=_QSHAPES,
    )(x, w0, hists_t)


def _layer_common(aggs, hist_ref, w_ref, b_ref):
    a = jnp.concatenate([q[...] for q in aggs], axis=1)
    deg_in = jnp.sum(hist_ref[1], axis=1, keepdims=True)
    nd = lax.rsqrt(jnp.maximum(deg_in, 1.0))
    h = jnp.dot(a * nd, w_ref[...], preferred_element_type=jnp.float32)
    return jnp.maximum(h + b_ref[...], 0.0)


def _tc_layer_body(a0, a1, a2, a3, hist_ref, w_ref, b_ref, *outs):
    h = _layer_common((a0, a1, a2, a3), hist_ref, w_ref, b_ref)
    deg_out = jnp.sum(hist_ref[0], axis=1, keepdims=True)
    ns = lax.rsqrt(jnp.maximum(deg_out, 1.0))
    _split_store(h * ns, outs)


def _tc_layer(aggs, hists_t, w, b):
    return pl.pallas_call(
        _tc_layer_body,
        grid=(N // BN,),
        in_specs=_QSPECS + [
            pl.BlockSpec((NSC, BN, NTEC), lambda i: (0, i, 0)),
            pl.BlockSpec((F, F), lambda i: (0, 0)),
            pl.BlockSpec((1, F), lambda i: (0, 0)),
        ],
        out_specs=_QSPECS,
        out_shape=_QSHAPES,
    )(*aggs, hists_t, w, b)


def _tc_final_body(a0, a1, a2, a3, hist_ref, w_ref, b_ref, out_ref):
    i = pl.program_id(0)

    @pl.when(i == 0)
    def _():
        out_ref[...] = jnp.full((1, F), -jnp.inf, jnp.float32)

    h = _layer_common((a0, a1, a2, a3), hist_ref, w_ref, b_ref)
    out_ref[...] = jnp.maximum(out_ref[...], jnp.max(h, axis=0, keepdims=True))


def _tc_final(aggs, hists_t, w, b):
    return pl.pallas_call(
        _tc_final_body,
        grid=(N // BN,),
        in_specs=_QSPECS + [
            pl.BlockSpec((NSC, BN, NTEC), lambda i: (0, i, 0)),
            pl.BlockSpec((F, F), lambda i: (0, 0)),
            pl.BlockSpec((1, F), lambda i: (0, 0)),
        ],
        out_specs=pl.BlockSpec((1, F), lambda i: (0, 0)),
        out_shape=jax.ShapeDtypeStruct((1, F), jnp.float32),
        compiler_params=pltpu.CompilerParams(
            dimension_semantics=("arbitrary",)),
    )(*aggs, hists_t, w, b)


@jax.jit
def kernel(x, edge_index, W0, W1, b1, W2, b2):
    src = edge_index[0]
    dst = edge_index[1]
    pad = jnp.full((E_PAD - E,), N, jnp.int32)
    src_p = jnp.concatenate([src, pad]).reshape(NTEC, NB, EDGE_BLK)
    dst_p = jnp.concatenate([dst, pad]).reshape(NTEC, NB, EDGE_BLK)

    hists = _sc_degree(src_p, dst_p)
    hists_t = hists.transpose(0, 2, 1)

    hs_q = _tc_embed(x, W0, hists_t)
    agg_q = _sc_agg(hs_q, src_p, dst_p)
    hs_q = _tc_layer(agg_q, hists_t, W1, b1.reshape(1, F))
    agg_q = _sc_agg(hs_q, src_p, dst_p)
    out = _tc_final(agg_q, hists_t, W2, b2.reshape(1, F))
    return out.reshape(F)
